# baseline (speedup 1.0000x reference)
import jax
import jax.numpy as jnp
from jax import lax
from jax.experimental import pallas as pl
from jax.experimental.pallas import tpu as pltpu
from jax.experimental.pallas import tpu_sc as plsc

NC = 2
NS = 16
L = 16
NW = NC * NS
CHUNK = 128
HID = 128
SEQ = 200
PAD_ID = 0
PAD_ROW = SEQ + CHUNK
PEXT_ROWS = 336


def _build(n_tok):
    per_w = n_tok // NW
    cpw = per_w // CHUNK
    assert cpw % 2 == 0
    mesh = plsc.VectorSubcoreMesh(core_axis_name="c", subcore_axis_name="s")

    def body(enc_ids, dec_ids, src_tab, trg_tab, pext_hbm, enc_out, dec_out,
             idx_big, pext, tok0, tok1, out0, out1,
             sem_t0, sem_t1, sem_o0, sem_o1):
        wid = lax.axis_index("s") * NC + lax.axis_index("c")
        tok = (tok0, tok1)
        out = (out0, out1)
        sem_t = (sem_t0, sem_t1)
        sem_o = (sem_o0, sem_o1)

        pltpu.sync_copy(pext_hbm, pext)

        for ids_hbm, tab_hbm, out_hbm in ((enc_ids, src_tab, enc_out),
                                          (dec_ids, trg_tab, dec_out)):
            pltpu.sync_copy(ids_hbm.at[pl.ds(wid * per_w, per_w)], idx_big)

            def issue(c, s):
                pltpu.async_copy(tab_hbm.at[idx_big.at[pl.ds(c * CHUNK, CHUNK)]],
                                 tok[s], sem_t[s])

            def consume(c, s):
                pltpu.make_async_copy(tab_hbm.at[idx_big.at[pl.ds(c * CHUNK,
                                                                  CHUNK)]],
                                      tok[s], sem_t[s]).wait()
                base = (wid * cpw + c) * CHUNK
                r0 = lax.rem(base, SEQ)

                @pl.when(c > 1)
                def _():
                    pltpu.make_async_copy(out[s], out_hbm.at[pl.ds(base, CHUNK)],
                                          sem_o[s]).wait()

                @plsc.parallel_loop(0, CHUNK, unroll=2)
                def _tok(i):
                    for j in range(HID // L):
                        sl = pl.ds(j * L, L)
                        out[s][i, sl] = tok[s][i, sl] + pext[r0 + i, sl]

                macc = None
                for g in range(CHUNK // L):
                    ids16 = idx_big[pl.ds(c * CHUNK + g * L, L)]
                    pm = jnp.where(ids16 == PAD_ID, 1, 0)
                    macc = pm if macc is None else macc + pm
                tot = macc[0]
                for k in range(1, L):
                    tot = tot + macc[k]

                @pl.when(tot > 0)
                def _():
                    @pl.loop(0, CHUNK // L)
                    def _grp(g):
                        ids16 = idx_big[pl.ds(c * CHUNK + g * L, L)]
                        for k in range(L):
                            @pl.when(ids16[k] == PAD_ID)
                            def _(k=k):
                                row = g * L + k
                                for j in range(HID // L):
                                    sl = pl.ds(j * L, L)
                                    out[s][row, sl] = (tok[s][row, sl]
                                                       + pext[PAD_ROW, sl])

                pltpu.async_copy(out[s], out_hbm.at[pl.ds(base, CHUNK)],
                                 sem_o[s])

            issue(0, 0)
            issue(1, 1)

            @pl.loop(0, cpw, step=2)
            def _chunks(c):
                consume(c, 0)

                @pl.when(c + 2 < cpw)
                def _():
                    issue(c + 2, 0)

                consume(c + 1, 1)

                @pl.when(c + 3 < cpw)
                def _():
                    issue(c + 3, 1)

            for s in (0, 1):
                pltpu.make_async_copy(out[s], out_hbm.at[pl.ds(0, CHUNK)],
                                      sem_o[s]).wait()

    return pl.kernel(
        body,
        out_type=(jax.ShapeDtypeStruct((n_tok, HID), jnp.float32),
                  jax.ShapeDtypeStruct((n_tok, HID), jnp.float32)),
        mesh=mesh,
        scratch_types=[
            pltpu.VMEM((n_tok // NW,), jnp.int32),
            pltpu.VMEM((PEXT_ROWS, HID), jnp.float32),
            pltpu.VMEM((CHUNK, HID), jnp.float32),
            pltpu.VMEM((CHUNK, HID), jnp.float32),
            pltpu.VMEM((CHUNK, HID), jnp.float32),
            pltpu.VMEM((CHUNK, HID), jnp.float32),
            pltpu.SemaphoreType.DMA,
            pltpu.SemaphoreType.DMA,
            pltpu.SemaphoreType.DMA,
            pltpu.SemaphoreType.DMA,
        ],
    )


def kernel(enc_ids, dec_ids, src_table, trg_table, pos_table):
    B, T = enc_ids.shape
    n_tok = B * T
    enc_flat = enc_ids.astype(jnp.int32).reshape(n_tok)
    dec_flat = dec_ids.astype(jnp.int32).reshape(n_tok)
    wrap = jnp.concatenate([pos_table[1:SEQ + 1], pos_table[1:CHUNK + 1],
                            pos_table[0:1],
                            jnp.zeros((PEXT_ROWS - PAD_ROW - 1, HID),
                                      jnp.float32)])
    enc_o, dec_o = _build(n_tok)(enc_flat, dec_flat, src_table, trg_table,
                                 wrap)
    return enc_o.reshape(B, T, HID), dec_o.reshape(B, T, HID)
